# baseline (speedup 1.0000x reference)
import functools

import jax
import jax.numpy as jnp
from jax import lax
from jax.experimental import pallas as pl
from jax.experimental.pallas import tpu as pltpu
from jax.experimental.pallas import tpu_sc as plsc

NC = 2
NS = 16
NW = NC * NS
L = 16


def _cdiv(a, b):
    return (a + b - 1) // b


def kernel(x, edge_index, relations, injection_node, node_batch,
           injection_node_batch, edge_attr):
    n, d = x.shape
    e = edge_index.shape[1]
    r, ed = relations.shape
    b = injection_node.shape[0]
    idt = edge_index.dtype
    en = e + n

    cx = _cdiv(n, NW)
    cx += (-cx) % 8

    mesh = plsc.VectorSubcoreMesh(core_axis_name="c", subcore_axis_name="s",
                                  num_cores=NC, num_subcores=NS)

    @functools.partial(
        pl.kernel,
        out_type=jax.ShapeDtypeStruct((n,), idt),
        mesh=mesh,
        compiler_params=pltpu.CompilerParams(needs_layout_passes=False),
        scratch_types=[
            pltpu.VMEM((cx,), jnp.int32),
            pltpu.VMEM((b,), jnp.int32),
            pltpu.VMEM((cx,), jnp.int32),
            pltpu.SemaphoreType.DMA,
            pltpu.SemaphoreType.DMA,
        ],
    )
    def sc_gather(nb_h, inb_h, src_h, nb_v, tbl_v, src_v, sem_s, sem_out):
        wid = lax.axis_index("s") * NC + lax.axis_index("c")
        s = jnp.minimum(wid * cx, n - cx)
        h_in = [
            pltpu.async_copy(nb_h.at[pl.ds(s, cx)], nb_v, sem_s),
            pltpu.async_copy(inb_h, tbl_v, sem_s),
        ]
        for h in h_in:
            h.wait()

        def srcfill(j, _):
            o = j * L
            idx = nb_v[pl.ds(o, L)]
            src_v[pl.ds(o, L)] = plsc.load_gather(tbl_v, [idx]) + n
            return 0
        lax.fori_loop(0, cx // L, srcfill, 0)

        pltpu.async_copy(src_v, src_h.at[pl.ds(s, cx)], sem_out).wait()

    src_row = sc_gather(node_batch, injection_node_batch)

    CR = 5000
    nxb = n // CR
    NG = _cdiv(n + b, CR)
    CRI = 4096
    CEI = 112640
    ea2 = edge_attr.reshape(1, ed)

    def a_body(x_ref, inj_ref, rel_ref, ea_ref, ei_any,
               o_ref, xinj_ref, relo_ref, rinj_ref, einj_ref, eio_any,
               sem):
        j = pl.program_id(0)

        @pl.when(j == 0)
        def _():
            pltpu.make_async_copy(
                ei_any, eio_any.at[:, pl.ds(0, e)], sem).start()

        pad = jnp.zeros((CR - b, d), x_ref.dtype)
        inj_full = jnp.concatenate([inj_ref[...], pad], axis=0)
        o_ref[...] = jnp.where(j < nxb, x_ref[...], inj_full)
        rowg = jax.lax.broadcasted_iota(jnp.int32, (CRI,), 0) + j * CRI
        xinj_ref[...] = jnp.where(rowg < n, 0, 1).astype(jnp.int32)
        relo_ref[...] = jnp.concatenate([rel_ref[...], ea_ref[...]], axis=0)
        ri = jax.lax.broadcasted_iota(jnp.int32, (r + 1,), 0)
        rinj_ref[...] = jnp.where(ri < r, 0, 1).astype(jnp.int32)
        colg = jax.lax.broadcasted_iota(jnp.int32, (CEI,), 0) + j * CEI
        einj_ref[...] = jnp.where(colg < e, 0, 1).astype(jnp.int32)

        @pl.when(j == NG - 1)
        def _():
            pltpu.make_async_copy(
                ei_any, eio_any.at[:, pl.ds(0, e)], sem).wait()

    x_out, x_inj, rel_out, r_inj, e_inj, eio_p = pl.pallas_call(
        a_body,
        grid=(NG,),
        in_specs=[
            pl.BlockSpec((CR, d), lambda i: (jnp.minimum(i, nxb - 1), 0)),
            pl.BlockSpec((b, d), lambda i: (0, 0)),
            pl.BlockSpec((r, ed), lambda i: (0, 0)),
            pl.BlockSpec((1, ed), lambda i: (0, 0)),
            pl.BlockSpec(memory_space=pl.ANY),
        ],
        out_specs=[
            pl.BlockSpec((CR, d), lambda i: (i, 0)),
            pl.BlockSpec((CRI,), lambda i: (i,)),
            pl.BlockSpec((r + 1, ed), lambda i: (0, 0)),
            pl.BlockSpec((r + 1,), lambda i: (0,)),
            pl.BlockSpec((CEI,), lambda i: (i,)),
            pl.BlockSpec(memory_space=pl.ANY),
        ],
        out_shape=[
            jax.ShapeDtypeStruct((n + b, d), x.dtype),
            jax.ShapeDtypeStruct((n + b,), jnp.int32),
            jax.ShapeDtypeStruct((r + 1, ed), relations.dtype),
            jax.ShapeDtypeStruct((r + 1,), jnp.int32),
            jax.ShapeDtypeStruct((en,), jnp.int32),
            jax.ShapeDtypeStruct((3, en), idt),
        ],
        scratch_shapes=[pltpu.SemaphoreType.DMA],
        compiler_params=pltpu.CompilerParams(
            dimension_semantics=("arbitrary",)),
    )(x, injection_node, relations, ea2, edge_index)

    def b_body(eio_in, src_any, eio_out, src_v, tail_v, sem):
        del eio_in
        pltpu.make_async_copy(src_any, src_v, sem).start()
        pltpu.make_async_copy(src_any, src_v, sem).wait()
        rowi = jax.lax.broadcasted_iota(idt, (3, n), 0)
        colg = jax.lax.broadcasted_iota(idt, (3, n), 1)
        srow = jnp.broadcast_to(src_v[...].reshape(1, n), (3, n))
        tail_v[...] = jnp.where(rowi == 0, srow,
                                jnp.where(rowi == 1,
                                          jnp.full((3, n), r, idt), colg))
        cp = pltpu.make_async_copy(tail_v, eio_out.at[:, pl.ds(e, n)], sem)
        cp.start()
        cp.wait()

    eio = pl.pallas_call(
        b_body,
        in_specs=[
            pl.BlockSpec(memory_space=pl.ANY),
            pl.BlockSpec(memory_space=pl.ANY),
        ],
        out_specs=pl.BlockSpec(memory_space=pl.ANY),
        out_shape=jax.ShapeDtypeStruct((3, en), idt),
        scratch_shapes=[
            pltpu.VMEM((n,), idt),
            pltpu.VMEM((3, n), idt),
            pltpu.SemaphoreType.DMA,
        ],
        input_output_aliases={0: 0},
    )(eio_p, src_row)

    return (x_out, eio, rel_out, x_inj, e_inj, r_inj)


# speedup vs baseline: 5.5351x; 5.5351x over previous
import functools

import jax
import jax.numpy as jnp
from jax import lax
from jax.experimental import pallas as pl
from jax.experimental.pallas import tpu as pltpu
from jax.experimental.pallas import tpu_sc as plsc

NC = 2
NS = 16
NW = NC * NS
L = 16


def _cdiv(a, b):
    return (a + b - 1) // b


def kernel(x, edge_index, relations, injection_node, node_batch,
           injection_node_batch, edge_attr):
    n, d = x.shape
    e = edge_index.shape[1]
    r, ed = relations.shape
    b = injection_node.shape[0]
    idt = edge_index.dtype
    en = e + n

    cx = _cdiv(n, NW)
    cx += (-cx) % 8

    mesh = plsc.VectorSubcoreMesh(core_axis_name="c", subcore_axis_name="s",
                                  num_cores=NC, num_subcores=NS)

    @functools.partial(
        pl.kernel,
        out_type=jax.ShapeDtypeStruct((n,), idt),
        mesh=mesh,
        compiler_params=pltpu.CompilerParams(needs_layout_passes=False),
        scratch_types=[
            pltpu.VMEM((cx,), jnp.int32),
            pltpu.VMEM((b,), jnp.int32),
            pltpu.VMEM((cx,), jnp.int32),
            pltpu.SemaphoreType.DMA,
            pltpu.SemaphoreType.DMA,
        ],
    )
    def sc_gather(nb_h, inb_h, src_h, nb_v, tbl_v, src_v, sem_s, sem_out):
        wid = lax.axis_index("s") * NC + lax.axis_index("c")
        s = jnp.minimum(wid * cx, n - cx)
        h_in = [
            pltpu.async_copy(nb_h.at[pl.ds(s, cx)], nb_v, sem_s),
            pltpu.async_copy(inb_h, tbl_v, sem_s),
        ]
        for h in h_in:
            h.wait()

        def srcfill(j, _):
            o = j * L
            idx = nb_v[pl.ds(o, L)]
            src_v[pl.ds(o, L)] = plsc.load_gather(tbl_v, [idx]) + n
            return 0
        lax.fori_loop(0, cx // L, srcfill, 0)

        pltpu.async_copy(src_v, src_h.at[pl.ds(s, cx)], sem_out).wait()

    src_row = sc_gather(node_batch, injection_node_batch)

    CR = 5000
    nxb = n // CR
    NG = _cdiv(n + b, CR)
    CRI = 4096
    CEI = 112640
    ea2 = edge_attr.reshape(1, ed)

    def a_body(x_ref, inj_ref, rel_ref, ea_ref,
               o_ref, xinj_ref, relo_ref, rinj_ref, einj_ref):
        j = pl.program_id(0)
        pad = jnp.zeros((CR - b, d), x_ref.dtype)
        inj_full = jnp.concatenate([inj_ref[...], pad], axis=0)
        o_ref[...] = jnp.where(j < nxb, x_ref[...], inj_full)
        rowg = jax.lax.broadcasted_iota(jnp.int32, (CRI,), 0) + j * CRI
        xinj_ref[...] = jnp.where(rowg < n, 0, 1).astype(jnp.int32)
        relo_ref[...] = jnp.concatenate([rel_ref[...], ea_ref[...]], axis=0)
        ri = jax.lax.broadcasted_iota(jnp.int32, (r + 1,), 0)
        rinj_ref[...] = jnp.where(ri < r, 0, 1).astype(jnp.int32)
        colg = jax.lax.broadcasted_iota(jnp.int32, (CEI,), 0) + j * CEI
        einj_ref[...] = jnp.where(colg < e, 0, 1).astype(jnp.int32)

    x_out, x_inj, rel_out, r_inj, e_inj = pl.pallas_call(
        a_body,
        grid=(NG,),
        in_specs=[
            pl.BlockSpec((CR, d), lambda i: (jnp.minimum(i, nxb - 1), 0)),
            pl.BlockSpec((b, d), lambda i: (0, 0)),
            pl.BlockSpec((r, ed), lambda i: (0, 0)),
            pl.BlockSpec((1, ed), lambda i: (0, 0)),
        ],
        out_specs=[
            pl.BlockSpec((CR, d), lambda i: (i, 0)),
            pl.BlockSpec((CRI,), lambda i: (i,)),
            pl.BlockSpec((r + 1, ed), lambda i: (0, 0)),
            pl.BlockSpec((r + 1,), lambda i: (0,)),
            pl.BlockSpec((CEI,), lambda i: (i,)),
        ],
        out_shape=[
            jax.ShapeDtypeStruct((n + b, d), x.dtype),
            jax.ShapeDtypeStruct((n + b,), jnp.int32),
            jax.ShapeDtypeStruct((r + 1, ed), relations.dtype),
            jax.ShapeDtypeStruct((r + 1,), jnp.int32),
            jax.ShapeDtypeStruct((en,), jnp.int32),
        ],
        compiler_params=pltpu.CompilerParams(
            dimension_semantics=("arbitrary",)),
    )(x, injection_node, relations, ea2)

    def c_body(ei_ref, o_ref):
        o_ref[:, pl.ds(0, e)] = ei_ref[...]

    eio_p = pl.pallas_call(
        c_body,
        in_specs=[pl.BlockSpec((3, e), lambda: (0, 0))],
        out_specs=pl.BlockSpec((3, en), lambda: (0, 0)),
        out_shape=jax.ShapeDtypeStruct((3, en), idt),
    )(edge_index)

    def b_body(eio_in, src_any, eio_out, src_v, tail_v, sem):
        del eio_in
        pltpu.make_async_copy(src_any, src_v, sem).start()
        pltpu.make_async_copy(src_any, src_v, sem).wait()
        rowi = jax.lax.broadcasted_iota(idt, (3, n), 0)
        colg = jax.lax.broadcasted_iota(idt, (3, n), 1)
        srow = jnp.broadcast_to(src_v[...].reshape(1, n), (3, n))
        tail_v[...] = jnp.where(rowi == 0, srow,
                                jnp.where(rowi == 1,
                                          jnp.full((3, n), r, idt), colg))
        cp = pltpu.make_async_copy(tail_v, eio_out.at[:, pl.ds(e, n)], sem)
        cp.start()
        cp.wait()

    eio = pl.pallas_call(
        b_body,
        in_specs=[
            pl.BlockSpec(memory_space=pl.ANY),
            pl.BlockSpec(memory_space=pl.ANY),
        ],
        out_specs=pl.BlockSpec(memory_space=pl.ANY),
        out_shape=jax.ShapeDtypeStruct((3, en), idt),
        scratch_shapes=[
            pltpu.VMEM((n,), idt),
            pltpu.VMEM((3, n), idt),
            pltpu.SemaphoreType.DMA,
        ],
        input_output_aliases={0: 0},
    )(eio_p, src_row)

    return (x_out, eio, rel_out, x_inj, e_inj, r_inj)


# speedup vs baseline: 6.1001x; 1.1021x over previous
import functools

import jax
import jax.numpy as jnp
from jax import lax
from jax.experimental import pallas as pl
from jax.experimental.pallas import tpu as pltpu
from jax.experimental.pallas import tpu_sc as plsc

NC = 2
NS = 16
NW = NC * NS
L = 16


def _cdiv(a, b):
    return (a + b - 1) // b


def kernel(x, edge_index, relations, injection_node, node_batch,
           injection_node_batch, edge_attr):
    n, d = x.shape
    e = edge_index.shape[1]
    r, ed = relations.shape
    b = injection_node.shape[0]
    idt = edge_index.dtype
    en = e + n

    cx = _cdiv(n, NW)
    cx += (-cx) % 8

    mesh = plsc.VectorSubcoreMesh(core_axis_name="c", subcore_axis_name="s",
                                  num_cores=NC, num_subcores=NS)

    @functools.partial(
        pl.kernel,
        out_type=jax.ShapeDtypeStruct((n,), idt),
        mesh=mesh,
        compiler_params=pltpu.CompilerParams(needs_layout_passes=False),
        scratch_types=[
            pltpu.VMEM((cx,), jnp.int32),
            pltpu.VMEM((b,), jnp.int32),
            pltpu.VMEM((cx,), jnp.int32),
            pltpu.SemaphoreType.DMA,
            pltpu.SemaphoreType.DMA,
        ],
    )
    def sc_gather(nb_h, inb_h, src_h, nb_v, tbl_v, src_v, sem_s, sem_out):
        wid = lax.axis_index("s") * NC + lax.axis_index("c")
        s = jnp.minimum(wid * cx, n - cx)
        h_in = [
            pltpu.async_copy(nb_h.at[pl.ds(s, cx)], nb_v, sem_s),
            pltpu.async_copy(inb_h, tbl_v, sem_s),
        ]
        for h in h_in:
            h.wait()

        def srcfill(j, _):
            o = j * L
            idx = nb_v[pl.ds(o, L)]
            src_v[pl.ds(o, L)] = plsc.load_gather(tbl_v, [idx]) + n
            return 0
        lax.fori_loop(0, cx // L, srcfill, 0)

        pltpu.async_copy(src_v, src_h.at[pl.ds(s, cx)], sem_out).wait()

    src_row = sc_gather(node_batch, injection_node_batch)

    CR = 5000
    nxb = n // CR
    NG = _cdiv(n + b, CR)
    CRI = 4096
    CEI = 112640
    ea2 = edge_attr.reshape(1, ed)

    def a_body(x_ref, inj_ref, rel_ref, ea_ref,
               o_ref, xinj_ref, relo_ref, rinj_ref, einj_ref):
        j = pl.program_id(0)
        pad = jnp.zeros((CR - b, d), x_ref.dtype)
        inj_full = jnp.concatenate([inj_ref[...], pad], axis=0)
        o_ref[...] = jnp.where(j < nxb, x_ref[...], inj_full)
        rowg = jax.lax.broadcasted_iota(jnp.int32, (CRI,), 0) + j * CRI
        xinj_ref[...] = jnp.where(rowg < n, 0, 1).astype(jnp.int32)
        relo_ref[...] = jnp.concatenate([rel_ref[...], ea_ref[...]], axis=0)
        ri = jax.lax.broadcasted_iota(jnp.int32, (r + 1,), 0)
        rinj_ref[...] = jnp.where(ri < r, 0, 1).astype(jnp.int32)
        colg = jax.lax.broadcasted_iota(jnp.int32, (CEI,), 0) + j * CEI
        einj_ref[...] = jnp.where(colg < e, 0, 1).astype(jnp.int32)

    x_out, x_inj, rel_out, r_inj, e_inj = pl.pallas_call(
        a_body,
        grid=(NG,),
        in_specs=[
            pl.BlockSpec((CR, d), lambda i: (jnp.minimum(i, nxb - 1), 0)),
            pl.BlockSpec((b, d), lambda i: (0, 0)),
            pl.BlockSpec((r, ed), lambda i: (0, 0)),
            pl.BlockSpec((1, ed), lambda i: (0, 0)),
        ],
        out_specs=[
            pl.BlockSpec((CR, d), lambda i: (i, 0)),
            pl.BlockSpec((CRI,), lambda i: (i,)),
            pl.BlockSpec((r + 1, ed), lambda i: (0, 0)),
            pl.BlockSpec((r + 1,), lambda i: (0,)),
            pl.BlockSpec((CEI,), lambda i: (i,)),
        ],
        out_shape=[
            jax.ShapeDtypeStruct((n + b, d), x.dtype),
            jax.ShapeDtypeStruct((n + b,), jnp.int32),
            jax.ShapeDtypeStruct((r + 1, ed), relations.dtype),
            jax.ShapeDtypeStruct((r + 1,), jnp.int32),
            jax.ShapeDtypeStruct((en,), jnp.int32),
        ],
        compiler_params=pltpu.CompilerParams(
            dimension_semantics=("arbitrary",)),
    )(x, injection_node, relations, ea2)

    def c_body(ei_ref, src_ref, o_ref):
        o_ref[:, pl.ds(0, e)] = ei_ref[...]
        rowi = jax.lax.broadcasted_iota(idt, (3, n), 0)
        colg = jax.lax.broadcasted_iota(idt, (3, n), 1)
        srow = jnp.broadcast_to(src_ref[...].reshape(1, n), (3, n))
        o_ref[:, pl.ds(e, n)] = jnp.where(
            rowi == 0, srow,
            jnp.where(rowi == 1, jnp.full((3, n), r, idt), colg))

    eio = pl.pallas_call(
        c_body,
        in_specs=[
            pl.BlockSpec((3, e), lambda: (0, 0)),
            pl.BlockSpec((n,), lambda: (0,)),
        ],
        out_specs=pl.BlockSpec((3, en), lambda: (0, 0)),
        out_shape=jax.ShapeDtypeStruct((3, en), idt),
    )(edge_index, src_row)

    return (x_out, eio, rel_out, x_inj, e_inj, r_inj)


# speedup vs baseline: 6.4232x; 1.0530x over previous
import functools

import jax
import jax.numpy as jnp
from jax import lax
from jax.experimental import pallas as pl
from jax.experimental.pallas import tpu as pltpu
from jax.experimental.pallas import tpu_sc as plsc

NC = 1
NS = 16
NW = NC * NS
L = 16


def _cdiv(a, b):
    return (a + b - 1) // b


def kernel(x, edge_index, relations, injection_node, node_batch,
           injection_node_batch, edge_attr):
    n, d = x.shape
    e = edge_index.shape[1]
    r, ed = relations.shape
    b = injection_node.shape[0]
    idt = edge_index.dtype
    en = e + n

    cx = _cdiv(n, NW)
    cx += (-cx) % 8

    mesh = plsc.VectorSubcoreMesh(core_axis_name="c", subcore_axis_name="s",
                                  num_cores=NC, num_subcores=NS)

    @functools.partial(
        pl.kernel,
        out_type=jax.ShapeDtypeStruct((n,), idt),
        mesh=mesh,
        compiler_params=pltpu.CompilerParams(needs_layout_passes=False),
        scratch_types=[
            pltpu.VMEM((cx,), jnp.int32),
            pltpu.VMEM((b,), jnp.int32),
            pltpu.VMEM((cx,), jnp.int32),
            pltpu.SemaphoreType.DMA,
            pltpu.SemaphoreType.DMA,
        ],
    )
    def sc_gather(nb_h, inb_h, src_h, nb_v, tbl_v, src_v, sem_s, sem_out):
        wid = lax.axis_index("s") * NC + lax.axis_index("c")
        s = jnp.minimum(wid * cx, n - cx)
        h_in = [
            pltpu.async_copy(nb_h.at[pl.ds(s, cx)], nb_v, sem_s),
            pltpu.async_copy(inb_h, tbl_v, sem_s),
        ]
        for h in h_in:
            h.wait()

        def srcfill(j, _):
            o = j * L
            idx = nb_v[pl.ds(o, L)]
            src_v[pl.ds(o, L)] = plsc.load_gather(tbl_v, [idx]) + n
            return 0
        lax.fori_loop(0, cx // L, srcfill, 0)

        pltpu.async_copy(src_v, src_h.at[pl.ds(s, cx)], sem_out).wait()

    src_row = sc_gather(node_batch, injection_node_batch)

    CR = 5000
    nxb = n // CR
    NG = _cdiv(n + b, CR)
    CRI = 4096
    CEI = 112640
    ea2 = edge_attr.reshape(1, ed)

    def a_body(x_ref, inj_ref, rel_ref, ea_ref,
               o_ref, xinj_ref, relo_ref, rinj_ref, einj_ref):
        j = pl.program_id(0)
        pad = jnp.zeros((CR - b, d), x_ref.dtype)
        inj_full = jnp.concatenate([inj_ref[...], pad], axis=0)
        o_ref[...] = jnp.where(j < nxb, x_ref[...], inj_full)
        rowg = jax.lax.broadcasted_iota(jnp.int32, (CRI,), 0) + j * CRI
        xinj_ref[...] = jnp.where(rowg < n, 0, 1).astype(jnp.int32)
        relo_ref[...] = jnp.concatenate([rel_ref[...], ea_ref[...]], axis=0)
        ri = jax.lax.broadcasted_iota(jnp.int32, (r + 1,), 0)
        rinj_ref[...] = jnp.where(ri < r, 0, 1).astype(jnp.int32)
        colg = jax.lax.broadcasted_iota(jnp.int32, (CEI,), 0) + j * CEI
        einj_ref[...] = jnp.where(colg < e, 0, 1).astype(jnp.int32)

    x_out, x_inj, rel_out, r_inj, e_inj = pl.pallas_call(
        a_body,
        grid=(NG,),
        in_specs=[
            pl.BlockSpec((CR, d), lambda i: (jnp.minimum(i, nxb - 1), 0)),
            pl.BlockSpec((b, d), lambda i: (0, 0)),
            pl.BlockSpec((r, ed), lambda i: (0, 0)),
            pl.BlockSpec((1, ed), lambda i: (0, 0)),
        ],
        out_specs=[
            pl.BlockSpec((CR, d), lambda i: (i, 0)),
            pl.BlockSpec((CRI,), lambda i: (i,)),
            pl.BlockSpec((r + 1, ed), lambda i: (0, 0)),
            pl.BlockSpec((r + 1,), lambda i: (0,)),
            pl.BlockSpec((CEI,), lambda i: (i,)),
        ],
        out_shape=[
            jax.ShapeDtypeStruct((n + b, d), x.dtype),
            jax.ShapeDtypeStruct((n + b,), jnp.int32),
            jax.ShapeDtypeStruct((r + 1, ed), relations.dtype),
            jax.ShapeDtypeStruct((r + 1,), jnp.int32),
            jax.ShapeDtypeStruct((en,), jnp.int32),
        ],
        compiler_params=pltpu.CompilerParams(
            dimension_semantics=("arbitrary",)),
    )(x, injection_node, relations, ea2)

    def c_body(ei_ref, src_ref, o_ref):
        o_ref[:, pl.ds(0, e)] = ei_ref[...]
        rowi = jax.lax.broadcasted_iota(idt, (3, n), 0)
        colg = jax.lax.broadcasted_iota(idt, (3, n), 1)
        srow = jnp.broadcast_to(src_ref[...].reshape(1, n), (3, n))
        o_ref[:, pl.ds(e, n)] = jnp.where(
            rowi == 0, srow,
            jnp.where(rowi == 1, jnp.full((3, n), r, idt), colg))

    eio = pl.pallas_call(
        c_body,
        in_specs=[
            pl.BlockSpec((3, e), lambda: (0, 0)),
            pl.BlockSpec((n,), lambda: (0,)),
        ],
        out_specs=pl.BlockSpec((3, en), lambda: (0, 0)),
        out_shape=jax.ShapeDtypeStruct((3, en), idt),
    )(edge_index, src_row)

    return (x_out, eio, rel_out, x_inj, e_inj, r_inj)
